# baseline (speedup 1.0000x reference)
import jax
import jax.numpy as jnp
import numpy as np
from jax import lax
from jax.experimental import pallas as pl
from jax.experimental.pallas import tpu as pltpu
from jax.experimental.pallas import tpu_sc as plsc

_B, _L, _D = 32, 576, 768


def _tf_rotl(x, r):
    return ((x << np.uint32(r)) | (x >> np.uint32(32 - r))).astype(np.uint32)


def _tf_uniform(seed, shape):
    n = int(np.prod(shape))
    k0 = np.uint32(seed >> 32)
    k1 = np.uint32(seed & 0xFFFFFFFF)
    ks = [k0, k1, (k0 ^ k1 ^ np.uint32(0x1BD11BDA)).astype(np.uint32)]
    rot = ([13, 15, 26, 6], [17, 29, 16, 24])
    x0 = (np.zeros(n, dtype=np.uint32) + ks[0]).astype(np.uint32)
    x1 = (np.arange(n, dtype=np.uint32) + ks[1]).astype(np.uint32)
    for i in range(5):
        for r in rot[i % 2]:
            x0 = (x0 + x1).astype(np.uint32)
            x1 = (_tf_rotl(x1, r) ^ x0).astype(np.uint32)
        x0 = (x0 + ks[(i + 1) % 3]).astype(np.uint32)
        x1 = (x1 + ks[(i + 2) % 3] + np.uint32(i + 1)).astype(np.uint32)
    bits = x0 ^ x1
    f = ((bits >> np.uint32(9)) | np.uint32(0x3F800000)).view(np.float32) - 1.0
    return f.reshape(shape)


_R = _tf_uniform(42, (_B, _L))

_NC, _NS = 2, 16
_NW = _NC * _NS

_CHUNK = 64
_NCHUNK = _L // _CHUNK
_NLANE = 16


def _rank_row(r):
    L = _L
    R = jnp.broadcast_to(r, (L, L))
    kk = lax.broadcasted_iota(jnp.int32, (L, L), 0)
    jj = lax.broadcasted_iota(jnp.int32, (L, L), 1)
    eye = kk == jj
    rcol = jnp.sum(jnp.where(eye, R, 0.0), axis=1, keepdims=True)
    C = jnp.broadcast_to(rcol, (L, L))
    less = (C < R).astype(jnp.int32)
    return jnp.sum(less, axis=0, keepdims=True)


def _indices_body(r_ref, idx_ref):
    L = _L
    rank = _rank_row(r_ref[0])
    ii = lax.broadcasted_iota(jnp.int32, (L, L), 0)
    jj = lax.broadcasted_iota(jnp.int32, (L, L), 1)
    hit = jnp.broadcast_to(rank, (L, L)) == ii
    perm_col = jnp.sum(jnp.where(hit, jj, 0), axis=1, keepdims=True)
    idx_ref[0] = jnp.broadcast_to(perm_col, (_L, _D))


def _tc_indices(r3):
    return pl.pallas_call(
        _indices_body,
        grid=(_B,),
        in_specs=[pl.BlockSpec((1, 1, _L), lambda b: (b, 0, 0))],
        out_specs=pl.BlockSpec((1, _L, _D), lambda b: (b, 0, 0)),
        out_shape=jax.ShapeDtypeStruct((_B, _L, _D), jnp.int32),
    )(r3)


def _sc_scatter_body(x_hbm, r_hbm, out_hbm,
                     r_v, idx_v, buf0, buf1, rsem0, rsem1, wsem0, wsem1):
    c = lax.axis_index("c")
    s = lax.axis_index("s")
    wid = s * _NC + c
    base = wid * _L
    pltpu.sync_copy(r_hbm.at[pl.ds(base, _L)], r_v)
    bufs = (buf0, buf1)
    rsems = (rsem0, rsem1)
    wsems = (wsem0, wsem1)
    reads = [None, None]
    writes = [None, None]
    reads[0] = pltpu.async_copy(x_hbm.at[pl.ds(base, _CHUNK)], buf0, rsem0)

    def _rank16(jbase):
        rj = r_v[pl.ds(jbase, _NLANE)]

        def kouter(ko, acc):
            kvec = r_v[pl.ds(ko * _NLANE, _NLANE)]
            for kk in range(_NLANE):
                rk = jnp.zeros((_NLANE,), jnp.float32) + kvec[kk]
                acc = acc + jnp.where(rk < rj, 1, 0)
            return acc

        return lax.fori_loop(0, _L // _NLANE, kouter,
                             jnp.zeros((_NLANE,), jnp.int32))

    for ch in range(_NCHUNK):
        for g in range(_CHUNK // _NLANE):
            jb = ch * _CHUNK + g * _NLANE
            idx_v[ch, pl.ds(g * _NLANE, _NLANE)] = _rank16(jb) + base
        nxt = ch + 1
        if nxt < _NCHUNK:
            p = nxt % 2
            if writes[p] is not None:
                writes[p].wait()
            reads[p] = pltpu.async_copy(
                x_hbm.at[pl.ds(base + nxt * _CHUNK, _CHUNK)], bufs[p],
                rsems[p])
        q = ch % 2
        reads[q].wait()
        writes[q] = pltpu.async_copy(bufs[q], out_hbm.at[idx_v.at[ch]],
                                     wsems[q])
    writes[0].wait()
    writes[1].wait()


def _sc_scatter(x_flat, r_flat):
    run = pl.kernel(
        _sc_scatter_body,
        mesh=plsc.VectorSubcoreMesh(core_axis_name="c", subcore_axis_name="s"),
        out_type=jax.ShapeDtypeStruct((_B * _L, _D), jnp.float32),
        scratch_types=[
            pltpu.VMEM((_L,), jnp.float32),
            pltpu.VMEM((_NCHUNK, _CHUNK), jnp.int32),
            pltpu.VMEM((_CHUNK, _D), jnp.float32),
            pltpu.VMEM((_CHUNK, _D), jnp.float32),
            pltpu.SemaphoreType.DMA,
            pltpu.SemaphoreType.DMA,
            pltpu.SemaphoreType.DMA,
            pltpu.SemaphoreType.DMA,
        ],
    )
    return run(x_flat, r_flat)


def kernel(x):
    B, L, D = x.shape
    r = jnp.asarray(_R)
    r3 = r.reshape(B, 1, L)
    shuffled = _sc_scatter(x.reshape(B * L, D), r.reshape(B * L))
    indices = _tc_indices(r3)
    return shuffled.reshape(B, L, D), indices
